# Initial kernel scaffold; baseline (speedup 1.0000x reference)
import jax
import jax.numpy as jnp
from jax.experimental import pallas as pl


def kernel(x, mask, toggle_gates):
    raise NotImplementedError("write your pallas kernel here")


# speedup vs baseline: 17.7649x; 17.7649x over previous
import jax
import jax.numpy as jnp
from jax.experimental import pallas as pl


def _gate_kernel(tg0_ref, tg1_ref, xb_ref, mask_ref, out_ref):
    i = pl.program_id(0)
    n = out_ref.shape[1]
    dense = tg0_ref[0, 0]
    alt = tg1_ref[0, 0][:, 0:1]
    pred = xb_ref[:, 0:1] > 0.5
    is_col0 = jax.lax.broadcasted_iota(jnp.int32, (n, n), 1) == 0
    use_alt = jnp.logical_and(jnp.logical_and(is_col0, pred), i < 3)
    gate = jnp.where(use_alt, alt, dense)
    val = jax.nn.sigmoid(gate)
    out_ref[0] = jnp.where(mask_ref[0], val, 0.0)


def kernel(x, mask, toggle_gates):
    c, _, n, _ = toggle_gates.shape
    xb = jnp.broadcast_to(x[:, None], (n, 128))
    mask3 = mask.reshape(c, n, n)
    out = pl.pallas_call(
        _gate_kernel,
        grid=(c,),
        in_specs=[
            pl.BlockSpec((1, 1, n, n), lambda i: (i, 0, 0, 0)),
            pl.BlockSpec((1, 1, n, 128), lambda i: (i, 1, 0, 0)),
            pl.BlockSpec((n, 128), lambda i: (0, 0)),
            pl.BlockSpec((1, n, n), lambda i: (i, 0, 0)),
        ],
        out_shape=jax.ShapeDtypeStruct((c, n, n), jnp.float32),
        out_specs=pl.BlockSpec((1, n, n), lambda i: (i, 0, 0)),
    )(toggle_gates, toggle_gates, xb, mask3)
    return out.reshape(-1)
